# baseline (speedup 1.0000x reference)
import functools

import jax
import jax.numpy as jnp
from jax import lax
from jax.experimental import pallas as pl
from jax.experimental.pallas import tpu as pltpu
from jax.experimental.pallas import tpu_sc as plsc

B = 1024
L = 50
V = 1000
LANES = 16

_NC = 2
_NS = 16
_NW = _NC * _NS
_ROWS_PER_W = B // _NW
_GROUPS = 2
_G_ROWS = _ROWS_PER_W // _GROUPS
_G_SPAN = _G_ROWS * V

_FULL_IDX_CHUNKS = L // LANES
_IDX_TAIL = L - LANES
_TAIL_LANES = L - _FULL_IDX_CHUNKS * LANES


def _sc_kernel(x_hbm, out_hbm, idx_v, acc_v, sem_in, sem_out):
    wid = lax.axis_index("s") * _NC + lax.axis_index("c")
    in_copy = pltpu.async_copy(
        x_hbm.at[pl.ds(wid * (_ROWS_PER_W * L), _ROWS_PER_W * L)], idx_v,
        sem_in)

    ones = jnp.ones((LANES,), jnp.float32)
    zeros = jnp.zeros((LANES,), jnp.float32)
    tail_mask = lax.iota(jnp.int32, LANES) >= (LANES - _TAIL_LANES)

    out_copies = []
    for g in range(_GROUPS):
        span = g * _G_SPAN

        @plsc.parallel_loop(span, span + _G_SPAN, step=LANES, unroll=8)
        def _zero(i):
            acc_v[pl.ds(i, LANES)] = zeros

        if g == 0:
            in_copy.wait()

        @plsc.parallel_loop(g * _G_ROWS, (g + 1) * _G_ROWS, step=1, unroll=4)
        def _scatter(r):
            acc_base = r * V
            idx_base = r * L
            for c in range(_FULL_IDX_CHUNKS):
                col = idx_v[pl.ds(idx_base + c * LANES, LANES)]
                plsc.addupdate_scatter(acc_v, [acc_base + col], ones)
            col = idx_v[pl.ds(idx_base + _IDX_TAIL, LANES)]
            plsc.addupdate_scatter(acc_v, [acc_base + col], ones,
                                   mask=tail_mask)

        out_copies.append(
            pltpu.async_copy(
                acc_v.at[pl.ds(span, _G_SPAN)],
                out_hbm.at[pl.ds(wid * (_ROWS_PER_W * V) + span, _G_SPAN)],
                sem_out))

    for c in out_copies:
        c.wait()


@jax.jit
def kernel(x):
    mesh = plsc.VectorSubcoreMesh(core_axis_name="c", subcore_axis_name="s")
    run = functools.partial(
        pl.kernel,
        mesh=mesh,
        compiler_params=pltpu.CompilerParams(
            use_tc_tiling_on_sc=False,
            needs_layout_passes=False,
            skip_device_barrier=True,
            disable_bounds_checks=True,
        ),
        out_type=jax.ShapeDtypeStruct((B * V,), jnp.float32),
        scratch_types=[
            pltpu.VMEM((_ROWS_PER_W * L,), jnp.int32),
            pltpu.VMEM((_ROWS_PER_W * V,), jnp.float32),
            pltpu.SemaphoreType.DMA,
            pltpu.SemaphoreType.DMA,
        ],
    )(_sc_kernel)
    return run(x.astype(jnp.int32).reshape(B * L)).reshape(B, V)


# speedup vs baseline: 1.0116x; 1.0116x over previous
import functools

import jax
import jax.numpy as jnp
from jax import lax
from jax.experimental import pallas as pl
from jax.experimental.pallas import tpu as pltpu
from jax.experimental.pallas import tpu_sc as plsc

B = 1024
L = 50
V = 1000
LANES = 16

_NC = 2
_NS = 16
_NW = _NC * _NS
_ROWS_PER_W = B // _NW
_GROUPS = 2
_G_ROWS = _ROWS_PER_W // _GROUPS
_G_SPAN = _G_ROWS * V

_FULL_IDX_CHUNKS = L // LANES
_IDX_TAIL = L - LANES
_TAIL_LANES = L - _FULL_IDX_CHUNKS * LANES


def _sc_kernel(x_hbm, out_hbm, idx_v, acc_v, sem_in, sem_out):
    wid = lax.axis_index("s") * _NC + lax.axis_index("c")
    in_copy = pltpu.async_copy(
        x_hbm.at[pl.ds(wid * (_ROWS_PER_W * L), _ROWS_PER_W * L)], idx_v,
        sem_in)

    ones = jnp.ones((LANES,), jnp.float32)
    zeros = jnp.zeros((LANES,), jnp.float32)
    tail_mask = lax.iota(jnp.int32, LANES) >= (LANES - _TAIL_LANES)

    out_copies = []
    for g in range(_GROUPS):
        span = g * _G_SPAN

        @plsc.parallel_loop(span, span + _G_SPAN, step=LANES, unroll=8)
        def _zero(i):
            acc_v[pl.ds(i, LANES)] = zeros

        if g == 0:
            in_copy.wait()

        @plsc.parallel_loop(g * _G_ROWS, (g + 1) * _G_ROWS, step=1, unroll=2)
        def _scatter(r):
            acc_base = r * V
            idx_base = r * L
            for c in range(_FULL_IDX_CHUNKS):
                col = idx_v[pl.ds(idx_base + c * LANES, LANES)]
                plsc.addupdate_scatter(acc_v, [acc_base + col], ones)
            col = idx_v[pl.ds(idx_base + _IDX_TAIL, LANES)]
            plsc.addupdate_scatter(acc_v, [acc_base + col], ones,
                                   mask=tail_mask)

        out_copies.append(
            pltpu.async_copy(
                acc_v.at[pl.ds(span, _G_SPAN)],
                out_hbm.at[pl.ds(wid * (_ROWS_PER_W * V) + span, _G_SPAN)],
                sem_out))

    for c in out_copies:
        c.wait()


@jax.jit
def kernel(x):
    mesh = plsc.VectorSubcoreMesh(core_axis_name="c", subcore_axis_name="s")
    run = functools.partial(
        pl.kernel,
        mesh=mesh,
        compiler_params=pltpu.CompilerParams(
            use_tc_tiling_on_sc=False,
            needs_layout_passes=False,
            skip_device_barrier=True,
            disable_bounds_checks=True,
        ),
        out_type=jax.ShapeDtypeStruct((B * V,), jnp.float32),
        scratch_types=[
            pltpu.VMEM((_ROWS_PER_W * L,), jnp.int32),
            pltpu.VMEM((_ROWS_PER_W * V,), jnp.float32),
            pltpu.SemaphoreType.DMA,
            pltpu.SemaphoreType.DMA,
        ],
    )(_sc_kernel)
    return run(x.astype(jnp.int32).reshape(B * L)).reshape(B, V)
